# baseline (speedup 1.0000x reference)
import jax
import jax.numpy as jnp
from jax.experimental import pallas as pl
from jax.experimental.pallas import tpu as pltpu

_B, _S, _D = 4, 4096, 2048
_NC = 64
_N = 192
_KC, _KQK, _KV = 8, 4, 6
_SBLK = 512


def _topk_renorm(v, k):
    n = v.shape[1]
    vi = v[:, :, None]
    vj = v[:, None, :]
    gt = (vj > vi).astype(jnp.float32)
    jj = jax.lax.broadcasted_iota(jnp.int32, (1, n, n), 2)
    ii = jax.lax.broadcasted_iota(jnp.int32, (1, n, n), 1)
    eq = ((vj == vi) & (jj < ii)).astype(jnp.float32)
    rank = jnp.sum(gt + eq, axis=2)
    sparse = jnp.where(rank < k, v, 0.0)
    return sparse / (jnp.sum(sparse, axis=1, keepdims=True) + 1e-8)


def _router_kernel(x_ref, imp_ref, w_ref, b_ref, emb_ref,
                   oc_ref, oq_ref, ok_ref, ov_ref, acc_ref):
    bi = pl.program_id(0)
    si = pl.program_id(1)
    nsb = pl.num_programs(1)

    @pl.when((bi == 0) & (si == 0))
    def _init():
        acc_ref[...] = jnp.zeros_like(acc_ref)

    x = x_ref[0]
    h = jax.lax.dot_general(x, w_ref[...], (((1,), (1,)), ((), ())),
                            preferred_element_type=jnp.float32)
    h = h + b_ref[...]

    emb = emb_ref[...]
    inv = jax.lax.rsqrt(jnp.maximum(jnp.sum(emb * emb, axis=1, keepdims=True),
                                    1e-24))
    emb_n = emb * inv
    logits = jax.lax.dot_general(h, emb_n, (((1,), (1,)), ((), ())),
                                 preferred_element_type=jnp.float32)

    probs = []
    for lo in (0, 64, 128):
        lg = logits[:, lo:lo + 64]
        m = jnp.max(lg, axis=1, keepdims=True)
        e = jnp.exp(lg - m)
        probs.append(e / jnp.sum(e, axis=1, keepdims=True))
    p = jnp.concatenate(probs, axis=1)

    imp = imp_ref[0, 0]
    contrib = jax.lax.dot_general(imp, p, (((1,), (0,)), ((), ())),
                                  preferred_element_type=jnp.float32)
    onehot = (jax.lax.broadcasted_iota(jnp.int32, (_B, 1), 0) == bi
              ).astype(jnp.float32)
    acc_ref[...] += onehot * contrib

    @pl.when((bi == _B - 1) & (si == nsb - 1))
    def _finish():
        acc = acc_ref[...]
        oc_ref[...] = _topk_renorm(acc[:, 0:64], _KC)
        q = _topk_renorm(acc[:, 64:128], _KQK)
        oq_ref[...] = q
        ok_ref[...] = q
        ov_ref[...] = _topk_renorm(acc[:, 128:192], _KV)


@jax.jit
def kernel(x, importance, W, b, neuron_emb):
    nsb = _S // _SBLK
    out_shape = tuple(jax.ShapeDtypeStruct((_B, _NC), jnp.float32)
                      for _ in range(4))
    outs = pl.pallas_call(
        _router_kernel,
        grid=(_B, nsb),
        in_specs=[
            pl.BlockSpec((1, _SBLK, _D), lambda bi, si: (bi, si, 0)),
            pl.BlockSpec((1, 1, 1, _SBLK), lambda bi, si: (bi, si, 0, 0)),
            pl.BlockSpec((_NC, _D), lambda bi, si: (0, 0)),
            pl.BlockSpec((1, _NC), lambda bi, si: (0, 0)),
            pl.BlockSpec((_N, _NC), lambda bi, si: (0, 0)),
        ],
        out_specs=tuple(pl.BlockSpec((_B, _NC), lambda bi, si: (0, 0))
                        for _ in range(4)),
        out_shape=out_shape,
        scratch_shapes=[pltpu.VMEM((_B, _N), jnp.float32)],
    )(x, importance.reshape(_B, nsb, 1, _SBLK), W, b.reshape(1, _NC),
      neuron_emb)
    return outs


# speedup vs baseline: 1.1134x; 1.1134x over previous
import jax
import jax.numpy as jnp
from jax.experimental import pallas as pl
from jax.experimental.pallas import tpu as pltpu

_B, _S, _D = 4, 4096, 2048
_NC = 64
_N = 192
_KC, _KQK, _KV = 8, 4, 6
_SBLK = 1024


def _topk_renorm(v, k):
    n = v.shape[1]
    vi = v[:, :, None]
    vj = v[:, None, :]
    gt = (vj > vi).astype(jnp.float32)
    jj = jax.lax.broadcasted_iota(jnp.int32, (1, n, n), 2)
    ii = jax.lax.broadcasted_iota(jnp.int32, (1, n, n), 1)
    eq = ((vj == vi) & (jj < ii)).astype(jnp.float32)
    rank = jnp.sum(gt + eq, axis=2)
    sparse = jnp.where(rank < k, v, 0.0)
    return sparse / (jnp.sum(sparse, axis=1, keepdims=True) + 1e-8)


def _router_kernel(x_ref, imp_ref, w_ref, b_ref, emb_ref,
                   oc_ref, oq_ref, ok_ref, ov_ref, acc_ref):
    bi = pl.program_id(0)
    si = pl.program_id(1)
    nsb = pl.num_programs(1)

    @pl.when((bi == 0) & (si == 0))
    def _init():
        acc_ref[...] = jnp.zeros_like(acc_ref)

    x = x_ref[0]
    h = jax.lax.dot_general(x, w_ref[...], (((1,), (1,)), ((), ())),
                            preferred_element_type=jnp.float32)
    h = h + b_ref[...]

    emb = emb_ref[...]
    inv = jax.lax.rsqrt(jnp.maximum(jnp.sum(emb * emb, axis=1, keepdims=True),
                                    1e-24))
    emb_n = emb * inv
    logits = jax.lax.dot_general(h, emb_n, (((1,), (1,)), ((), ())),
                                 preferred_element_type=jnp.float32)

    probs = []
    for lo in (0, 64, 128):
        lg = logits[:, lo:lo + 64]
        m = jnp.max(lg, axis=1, keepdims=True)
        e = jnp.exp(lg - m)
        probs.append(e / jnp.sum(e, axis=1, keepdims=True))
    p = jnp.concatenate(probs, axis=1)

    imp = imp_ref[0, 0]
    contrib = jax.lax.dot_general(imp, p, (((1,), (0,)), ((), ())),
                                  preferred_element_type=jnp.float32)
    onehot = (jax.lax.broadcasted_iota(jnp.int32, (_B, 1), 0) == bi
              ).astype(jnp.float32)
    acc_ref[...] += onehot * contrib

    @pl.when((bi == _B - 1) & (si == nsb - 1))
    def _finish():
        acc = acc_ref[...]
        oc_ref[...] = _topk_renorm(acc[:, 0:64], _KC)
        q = _topk_renorm(acc[:, 64:128], _KQK)
        oq_ref[...] = q
        ok_ref[...] = q
        ov_ref[...] = _topk_renorm(acc[:, 128:192], _KV)


@jax.jit
def kernel(x, importance, W, b, neuron_emb):
    nsb = _S // _SBLK
    out_shape = tuple(jax.ShapeDtypeStruct((_B, _NC), jnp.float32)
                      for _ in range(4))
    outs = pl.pallas_call(
        _router_kernel,
        grid=(_B, nsb),
        in_specs=[
            pl.BlockSpec((1, _SBLK, _D), lambda bi, si: (bi, si, 0)),
            pl.BlockSpec((1, 1, 1, _SBLK), lambda bi, si: (bi, si, 0, 0)),
            pl.BlockSpec((_NC, _D), lambda bi, si: (0, 0)),
            pl.BlockSpec((1, _NC), lambda bi, si: (0, 0)),
            pl.BlockSpec((_N, _NC), lambda bi, si: (0, 0)),
        ],
        out_specs=tuple(pl.BlockSpec((_B, _NC), lambda bi, si: (0, 0))
                        for _ in range(4)),
        out_shape=out_shape,
        scratch_shapes=[pltpu.VMEM((_B, _N), jnp.float32)],
    )(x, importance.reshape(_B, nsb, 1, _SBLK), W, b.reshape(1, _NC),
      neuron_emb)
    return outs


# speedup vs baseline: 1.1430x; 1.0266x over previous
import jax
import jax.numpy as jnp
from jax.experimental import pallas as pl
from jax.experimental.pallas import tpu as pltpu

_B, _S, _D = 4, 4096, 2048
_NC = 64
_N = 192
_KC, _KQK, _KV = 8, 4, 6
_SBLK = 2048


def _topk_renorm(v, k):
    n = v.shape[1]
    vi = v[:, :, None]
    vj = v[:, None, :]
    gt = (vj > vi).astype(jnp.float32)
    jj = jax.lax.broadcasted_iota(jnp.int32, (1, n, n), 2)
    ii = jax.lax.broadcasted_iota(jnp.int32, (1, n, n), 1)
    eq = ((vj == vi) & (jj < ii)).astype(jnp.float32)
    rank = jnp.sum(gt + eq, axis=2)
    sparse = jnp.where(rank < k, v, 0.0)
    return sparse / (jnp.sum(sparse, axis=1, keepdims=True) + 1e-8)


def _router_kernel(x_ref, imp_ref, w_ref, b_ref, emb_ref,
                   oc_ref, oq_ref, ok_ref, ov_ref, acc_ref):
    bi = pl.program_id(0)
    si = pl.program_id(1)
    nsb = pl.num_programs(1)

    @pl.when((bi == 0) & (si == 0))
    def _init():
        acc_ref[...] = jnp.zeros_like(acc_ref)

    x = x_ref[0]
    h = jax.lax.dot_general(x, w_ref[...], (((1,), (1,)), ((), ())),
                            preferred_element_type=jnp.float32)
    h = h + b_ref[...]

    emb = emb_ref[...]
    inv = jax.lax.rsqrt(jnp.maximum(jnp.sum(emb * emb, axis=1, keepdims=True),
                                    1e-24))
    emb_n = emb * inv
    logits = jax.lax.dot_general(h, emb_n, (((1,), (1,)), ((), ())),
                                 preferred_element_type=jnp.float32)

    probs = []
    for lo in (0, 64, 128):
        lg = logits[:, lo:lo + 64]
        m = jnp.max(lg, axis=1, keepdims=True)
        e = jnp.exp(lg - m)
        probs.append(e / jnp.sum(e, axis=1, keepdims=True))
    p = jnp.concatenate(probs, axis=1)

    imp = imp_ref[0, 0]
    contrib = jax.lax.dot_general(imp, p, (((1,), (0,)), ((), ())),
                                  preferred_element_type=jnp.float32)
    onehot = (jax.lax.broadcasted_iota(jnp.int32, (_B, 1), 0) == bi
              ).astype(jnp.float32)
    acc_ref[...] += onehot * contrib

    @pl.when((bi == _B - 1) & (si == nsb - 1))
    def _finish():
        acc = acc_ref[...]
        oc_ref[...] = _topk_renorm(acc[:, 0:64], _KC)
        q = _topk_renorm(acc[:, 64:128], _KQK)
        oq_ref[...] = q
        ok_ref[...] = q
        ov_ref[...] = _topk_renorm(acc[:, 128:192], _KV)


@jax.jit
def kernel(x, importance, W, b, neuron_emb):
    nsb = _S // _SBLK
    out_shape = tuple(jax.ShapeDtypeStruct((_B, _NC), jnp.float32)
                      for _ in range(4))
    outs = pl.pallas_call(
        _router_kernel,
        grid=(_B, nsb),
        in_specs=[
            pl.BlockSpec((1, _SBLK, _D), lambda bi, si: (bi, si, 0)),
            pl.BlockSpec((1, 1, 1, _SBLK), lambda bi, si: (bi, si, 0, 0)),
            pl.BlockSpec((_NC, _D), lambda bi, si: (0, 0)),
            pl.BlockSpec((1, _NC), lambda bi, si: (0, 0)),
            pl.BlockSpec((_N, _NC), lambda bi, si: (0, 0)),
        ],
        out_specs=tuple(pl.BlockSpec((_B, _NC), lambda bi, si: (0, 0))
                        for _ in range(4)),
        out_shape=out_shape,
        scratch_shapes=[pltpu.VMEM((_B, _N), jnp.float32)],
    )(x, importance.reshape(_B, nsb, 1, _SBLK), W, b.reshape(1, _NC),
      neuron_emb)
    return outs


# speedup vs baseline: 1.6217x; 1.4189x over previous
import jax
import jax.numpy as jnp
from jax.experimental import pallas as pl
from jax.experimental.pallas import tpu as pltpu

_B, _S, _D = 4, 4096, 2048
_NC = 64
_N = 192
_KC, _KQK, _KV = 8, 4, 6
_SBLK = 2048


def _topk_renorm(v, k):
    n = v.shape[1]
    vi = v[:, :, None]
    vj = v[:, None, :]
    gt = (vj > vi).astype(jnp.float32)
    jj = jax.lax.broadcasted_iota(jnp.int32, (1, n, n), 2)
    ii = jax.lax.broadcasted_iota(jnp.int32, (1, n, n), 1)
    eq = ((vj == vi) & (jj < ii)).astype(jnp.float32)
    rank = jnp.sum(gt + eq, axis=2)
    sparse = jnp.where(rank < k, v, 0.0)
    return sparse / (jnp.sum(sparse, axis=1, keepdims=True) + 1e-8)


def _router_kernel(x_ref, imp_ref, w_ref, b_ref, emb_ref,
                   oc_ref, oq_ref, ok_ref, ov_ref, acc_ref):
    bi = pl.program_id(0)
    si = pl.program_id(1)
    nsb = pl.num_programs(1)

    @pl.when((bi == 0) & (si == 0))
    def _init():
        acc_ref[...] = jnp.zeros_like(acc_ref)

    x = x_ref[0]
    ht = jax.lax.dot_general(w_ref[...], x, (((1,), (1,)), ((), ())),
                             preferred_element_type=jnp.float32)
    ht = ht + b_ref[...]

    emb = emb_ref[...]
    inv = jax.lax.rsqrt(jnp.maximum(jnp.sum(emb * emb, axis=1, keepdims=True),
                                    1e-24))
    emb_n = emb * inv
    lgt = jax.lax.dot_general(emb_n, ht, (((1,), (0,)), ((), ())),
                              preferred_element_type=jnp.float32)

    imp = imp_ref[0, 0]
    qs = []
    es = []
    for g in range(3):
        lg = lgt[64 * g:64 * (g + 1), :]
        m = jnp.max(lg, axis=0, keepdims=True)
        e = jnp.exp(lg - m)
        d = jnp.sum(e, axis=0, keepdims=True)
        qs.append(imp / d)
        es.append(e)
    e_full = jnp.concatenate(es, axis=0)
    q3 = jnp.concatenate(qs, axis=0)
    cf = jax.lax.dot_general(e_full, q3, (((1,), (1,)), ((), ())),
                             preferred_element_type=jnp.float32)
    grp = jax.lax.broadcasted_iota(jnp.int32, (_N, 3), 0) // 64
    gid = jax.lax.broadcasted_iota(jnp.int32, (_N, 3), 1)
    contrib = jnp.sum(jnp.where(grp == gid, cf, 0.0), axis=1,
                      keepdims=True)
    onehot = (jax.lax.broadcasted_iota(jnp.int32, (1, _B), 1) == bi
              ).astype(jnp.float32)
    acc_ref[...] += contrib * onehot

    @pl.when((bi == _B - 1) & (si == nsb - 1))
    def _finish():
        acc = acc_ref[...].T
        oc_ref[...] = _topk_renorm(acc[:, 0:64], _KC)
        q = _topk_renorm(acc[:, 64:128], _KQK)
        oq_ref[...] = q
        ok_ref[...] = q
        ov_ref[...] = _topk_renorm(acc[:, 128:192], _KV)


@jax.jit
def kernel(x, importance, W, b, neuron_emb):
    nsb = _S // _SBLK
    out_shape = tuple(jax.ShapeDtypeStruct((_B, _NC), jnp.float32)
                      for _ in range(4))
    outs = pl.pallas_call(
        _router_kernel,
        grid=(_B, nsb),
        in_specs=[
            pl.BlockSpec((1, _SBLK, _D), lambda bi, si: (bi, si, 0)),
            pl.BlockSpec((1, 1, 1, _SBLK), lambda bi, si: (bi, si, 0, 0)),
            pl.BlockSpec((_NC, _D), lambda bi, si: (0, 0)),
            pl.BlockSpec((_NC, 1), lambda bi, si: (0, 0)),
            pl.BlockSpec((_N, _NC), lambda bi, si: (0, 0)),
        ],
        out_specs=tuple(pl.BlockSpec((_B, _NC), lambda bi, si: (0, 0))
                        for _ in range(4)),
        out_shape=out_shape,
        scratch_shapes=[pltpu.VMEM((_N, _B), jnp.float32)],
    )(x, importance.reshape(_B, nsb, 1, _SBLK), W, b.reshape(_NC, 1),
      neuron_emb)
    return outs
